# baseline (speedup 1.0000x reference)
import functools

import jax
import jax.numpy as jnp
from jax import lax
from jax.experimental import pallas as pl
from jax.experimental.pallas import tpu as pltpu
from jax.experimental.pallas import tpu_sc as plsc

VOCAB_N = 1000000
D = 64
HID = 128
CLS = 2
B = 4096
HIST = 200

NC, NS = 2, 16
NW = NC * NS
ROWS = B // NW
C0 = 128
C1 = HIST - C0
C1P = 80
HALF = ROWS // 2
L = 16
INV = 1.0 / HIST


def _make_pool_kernel():
    mesh = plsc.VectorSubcoreMesh(core_axis_name="c", subcore_axis_name="s")

    @functools.partial(
        pl.kernel,
        mesh=mesh,
        out_type=jax.ShapeDtypeStruct((B, D), jnp.float32),
        scratch_types=[
            pltpu.VMEM((ROWS, HIST), jnp.int32),
            pltpu.VMEM((C0, D), jnp.float32),
            pltpu.VMEM((C1P, D), jnp.float32),
            pltpu.VMEM((C0, D), jnp.float32),
            pltpu.VMEM((C1P, D), jnp.float32),
            pltpu.VMEM((ROWS, D), jnp.float32),
            pltpu.SemaphoreType.DMA,
            pltpu.SemaphoreType.DMA,
            pltpu.SemaphoreType.DMA,
            pltpu.SemaphoreType.DMA,
        ],
        compiler_params=pltpu.CompilerParams(use_tc_tiling_on_sc=False),
    )
    def pool(x_hbm, emb_hbm, out_hbm, idx_v, b0, b1, b2, b3, acc_v,
             s0, s1, s2, s3):
        wid = lax.axis_index("s") * NC + lax.axis_index("c")
        base = wid * ROWS
        pltpu.sync_copy(x_hbm.at[pl.ds(base, ROWS)], idx_v)

        zrow = jnp.zeros((L,), jnp.float32)
        for buf in (b1, b3):
            def zero_row(r, _, buf=buf):
                for k in range(D // L):
                    buf[r, pl.ds(k * L, L)] = zrow
                return 0
            lax.fori_loop(C1, C1P, zero_row, 0)

        def fire0(r, buf, sem):
            pltpu.make_async_copy(
                emb_hbm.at[idx_v.at[r, pl.ds(0, C0)]], buf, sem).start()

        def fire1(r, buf, sem):
            pltpu.make_async_copy(
                emb_hbm.at[idx_v.at[r, pl.ds(C0, C1)]],
                buf.at[pl.ds(0, C1)], sem).start()

        def wait0(buf, sem):
            pltpu.make_async_copy(
                emb_hbm.at[idx_v.at[0, pl.ds(0, C0)]], buf, sem).wait()

        def wait1(buf, sem):
            pltpu.make_async_copy(
                emb_hbm.at[idx_v.at[0, pl.ds(C0, C1)]],
                buf.at[pl.ds(0, C1)], sem).wait()

        def accum(buf, nrows, acc):
            def body(j, a):
                return (a[0] + buf[j, pl.ds(0, L)],
                        a[1] + buf[j, pl.ds(L, L)],
                        a[2] + buf[j, pl.ds(2 * L, L)],
                        a[3] + buf[j, pl.ds(3 * L, L)])
            return lax.fori_loop(0, nrows, body, acc)

        def store_row(r, a):
            acc_v[r, pl.ds(0, L)] = a[0] * INV
            acc_v[r, pl.ds(L, L)] = a[1] * INV
            acc_v[r, pl.ds(2 * L, L)] = a[2] * INV
            acc_v[r, pl.ds(3 * L, L)] = a[3] * INV

        zeros = (jnp.zeros((L,), jnp.float32),) * 4

        fire0(0, b0, s0)
        fire1(0, b1, s1)
        fire0(1, b2, s2)
        fire1(1, b3, s3)

        def pair_body(bb, carry):
            r = 2 * bb
            not_last = bb < HALF - 1

            wait0(b0, s0)
            acc = accum(b0, C0, zeros)

            @pl.when(not_last)
            def _():
                fire0(r + 2, b0, s0)

            wait1(b1, s1)
            acc = accum(b1, C1P, acc)

            @pl.when(not_last)
            def _():
                fire1(r + 2, b1, s1)

            store_row(r, acc)

            wait0(b2, s2)
            acc2 = accum(b2, C0, zeros)

            @pl.when(not_last)
            def _():
                fire0(r + 3, b2, s2)

            wait1(b3, s3)
            acc2 = accum(b3, C1P, acc2)

            @pl.when(not_last)
            def _():
                fire1(r + 3, b3, s3)

            store_row(r + 1, acc2)
            return carry

        lax.fori_loop(0, HALF, pair_body, 0)
        pltpu.sync_copy(acc_v, out_hbm.at[pl.ds(base, ROWS)])

    return pool


_pool = _make_pool_kernel()


def _mlp_body(x_ref, w1t_ref, b1_ref, w2t_ref, b2_ref, o_ref):
    h = jnp.dot(x_ref[...], w1t_ref[...], preferred_element_type=jnp.float32)
    h = jnp.maximum(h + b1_ref[...], 0.0)
    o_ref[...] = (jnp.dot(h, w2t_ref[...], preferred_element_type=jnp.float32)
                  + b2_ref[...])


def kernel(x_in, emb, W1, b1, W2, b2):
    pooled = _pool(x_in, emb)
    logits = pl.pallas_call(
        _mlp_body,
        out_shape=jax.ShapeDtypeStruct((B, CLS), jnp.float32),
    )(pooled, W1.T, b1.reshape(1, HID), W2.T, b2.reshape(1, CLS))
    return logits


# speedup vs baseline: 1.0118x; 1.0118x over previous
import functools

import jax
import jax.numpy as jnp
from jax import lax
from jax.experimental import pallas as pl
from jax.experimental.pallas import tpu as pltpu
from jax.experimental.pallas import tpu_sc as plsc

VOCAB_N = 1000000
D = 64
HID = 128
CLS = 2
B = 4096
HIST = 200

NC, NS = 2, 16
NW = NC * NS
ROWS = B // NW
C0 = 128
C1 = HIST - C0
C1P = 80
HALF = ROWS // 2
L = 16
INV = 1.0 / HIST


def _make_pool_kernel():
    mesh = plsc.VectorSubcoreMesh(core_axis_name="c", subcore_axis_name="s")

    @functools.partial(
        pl.kernel,
        mesh=mesh,
        out_type=jax.ShapeDtypeStruct((B, D), jnp.float32),
        scratch_types=[
            pltpu.VMEM((ROWS, HIST), jnp.int32),
            pltpu.VMEM((C0, D), jnp.float32),
            pltpu.VMEM((C1P, D), jnp.float32),
            pltpu.VMEM((C0, D), jnp.float32),
            pltpu.VMEM((C1P, D), jnp.float32),
            pltpu.VMEM((ROWS, D), jnp.float32),
            pltpu.SemaphoreType.DMA,
            pltpu.SemaphoreType.DMA,
            pltpu.SemaphoreType.DMA,
            pltpu.SemaphoreType.DMA,
        ],
        compiler_params=pltpu.CompilerParams(use_tc_tiling_on_sc=False),
    )
    def pool(x_hbm, emb_hbm, out_hbm, idx_v, b0, b1, b2, b3, acc_v,
             s0, s1, s2, s3):
        wid = lax.axis_index("s") * NC + lax.axis_index("c")
        base = wid * ROWS
        pltpu.sync_copy(x_hbm.at[pl.ds(base, ROWS)], idx_v)

        zrow = jnp.zeros((L,), jnp.float32)
        for buf in (b1, b3):
            def zero_row(r, _, buf=buf):
                for k in range(D // L):
                    buf[r, pl.ds(k * L, L)] = zrow
                return 0
            lax.fori_loop(C1, C1P, zero_row, 0)

        def fire0(r, buf, sem):
            pltpu.make_async_copy(
                emb_hbm.at[idx_v.at[r, pl.ds(0, C0)]], buf, sem).start()

        def fire1(r, buf, sem):
            pltpu.make_async_copy(
                emb_hbm.at[idx_v.at[r, pl.ds(C0, C1)]],
                buf.at[pl.ds(0, C1)], sem).start()

        def wait0(buf, sem):
            pltpu.make_async_copy(
                emb_hbm.at[idx_v.at[0, pl.ds(0, C0)]], buf, sem).wait()

        def wait1(buf, sem):
            pltpu.make_async_copy(
                emb_hbm.at[idx_v.at[0, pl.ds(C0, C1)]],
                buf.at[pl.ds(0, C1)], sem).wait()

        def accum(buf, nrows, acc):
            def body(g, a):
                for k in range(8):
                    j = g * 8 + k
                    a = (a[0] + buf[j, pl.ds(0, L)],
                         a[1] + buf[j, pl.ds(L, L)],
                         a[2] + buf[j, pl.ds(2 * L, L)],
                         a[3] + buf[j, pl.ds(3 * L, L)])
                return a
            return lax.fori_loop(0, nrows // 8, body, acc)

        def store_row(r, a):
            acc_v[r, pl.ds(0, L)] = a[0] * INV
            acc_v[r, pl.ds(L, L)] = a[1] * INV
            acc_v[r, pl.ds(2 * L, L)] = a[2] * INV
            acc_v[r, pl.ds(3 * L, L)] = a[3] * INV

        zeros = (jnp.zeros((L,), jnp.float32),) * 4

        fire0(0, b0, s0)
        fire1(0, b1, s1)
        fire0(1, b2, s2)
        fire1(1, b3, s3)

        def pair_body(bb, carry):
            r = 2 * bb
            not_last = bb < HALF - 1

            wait0(b0, s0)
            acc = accum(b0, C0, zeros)

            @pl.when(not_last)
            def _():
                fire0(r + 2, b0, s0)

            wait1(b1, s1)
            acc = accum(b1, C1P, acc)

            @pl.when(not_last)
            def _():
                fire1(r + 2, b1, s1)

            store_row(r, acc)

            wait0(b2, s2)
            acc2 = accum(b2, C0, zeros)

            @pl.when(not_last)
            def _():
                fire0(r + 3, b2, s2)

            wait1(b3, s3)
            acc2 = accum(b3, C1P, acc2)

            @pl.when(not_last)
            def _():
                fire1(r + 3, b3, s3)

            store_row(r + 1, acc2)
            return carry

        lax.fori_loop(0, HALF, pair_body, 0)
        pltpu.sync_copy(acc_v, out_hbm.at[pl.ds(base, ROWS)])

    return pool


_pool = _make_pool_kernel()


def _mlp_body(x_ref, w1t_ref, b1_ref, w2t_ref, b2_ref, o_ref):
    h = jnp.dot(x_ref[...], w1t_ref[...], preferred_element_type=jnp.float32)
    h = jnp.maximum(h + b1_ref[...], 0.0)
    o_ref[...] = (jnp.dot(h, w2t_ref[...], preferred_element_type=jnp.float32)
                  + b2_ref[...])


def kernel(x_in, emb, W1, b1, W2, b2):
    pooled = _pool(x_in, emb)
    logits = pl.pallas_call(
        _mlp_body,
        out_shape=jax.ShapeDtypeStruct((B, CLS), jnp.float32),
    )(pooled, W1.T, b1.reshape(1, HID), W2.T, b2.reshape(1, CLS))
    return logits
